# Initial kernel scaffold; baseline (speedup 1.0000x reference)
import jax
import jax.numpy as jnp
from jax.experimental import pallas as pl


def kernel(tag, table):
    raise NotImplementedError("write your pallas kernel here")


# speedup vs baseline: 1.0153x; 1.0153x over previous
import functools
import math

import jax
import jax.numpy as jnp
from jax import lax
from jax.experimental import pallas as pl
from jax.experimental.pallas import tpu as pltpu
from jax.experimental.pallas import tpu_sc as plsc

C_DIM = 32
IDX_W = 128
SCALE = 1.0 / math.sqrt(C_DIM)


@functools.partial(jax.jit, static_argnames=("n_rows",))
def _gather_scaled(idx, table, n_rows):
    info = plsc.get_sparse_core_info()
    nw = info.num_cores * info.num_subcores
    rows_per_w = n_rows // nw
    G = 8
    n_groups = rows_per_w // G
    gb = G * IDX_W

    mesh = plsc.VectorSubcoreMesh(core_axis_name="c", subcore_axis_name="s")

    @functools.partial(
        pl.kernel,
        mesh=mesh,
        out_type=jax.ShapeDtypeStruct((n_rows * IDX_W, C_DIM), jnp.float32),
        scratch_types=[
            pltpu.VMEM((G, IDX_W), jnp.int32),
            pltpu.VMEM((gb, C_DIM), jnp.float32),
            pltpu.SemaphoreType.DMA,
        ],
        compiler_params=pltpu.CompilerParams(use_tc_tiling_on_sc=False),
    )
    def k(idx_hbm, table_hbm, out_hbm, idx_v, rows_v, sem):
        wid = lax.axis_index("s") * info.num_cores + lax.axis_index("c")
        base = wid * rows_per_w

        def group(g, carry):
            r0 = base + g * G
            pltpu.sync_copy(idx_hbm.at[pl.ds(r0, G), :], idx_v)
            copies = [
                pltpu.async_copy(
                    table_hbm.at[idx_v.at[j]],
                    rows_v.at[pl.ds(j * IDX_W, IDX_W)],
                    sem,
                )
                for j in range(G)
            ]
            for c in copies:
                c.wait()

            def scale_row(i, c):
                rows_v[i, 0:16] = rows_v[i, 0:16] * SCALE
                rows_v[i, 16:32] = rows_v[i, 16:32] * SCALE
                return c

            lax.fori_loop(0, gb, scale_row, 0, unroll=4)
            pltpu.sync_copy(rows_v, out_hbm.at[pl.ds(r0 * IDX_W, gb)])
            return carry

        lax.fori_loop(0, n_groups, group, 0)

    return k(idx, table)


def kernel(tag, table):
    b, s = tag.shape
    n = b * s
    idx = tag.reshape(n // IDX_W, IDX_W).astype(jnp.int32)
    out = _gather_scaled(idx, table, n // IDX_W)
    return out.reshape(b, s, C_DIM)


# speedup vs baseline: 1.5543x; 1.5308x over previous
import functools
import math

import jax
import jax.numpy as jnp
from jax import lax
from jax.experimental import pallas as pl
from jax.experimental.pallas import tpu as pltpu
from jax.experimental.pallas import tpu_sc as plsc

C_DIM = 32
SCALE = 1.0 / math.sqrt(C_DIM)


@jax.jit
def _emb_lookup(tag, table):
    T, S = tag.shape
    info = plsc.get_sparse_core_info()
    nw = info.num_cores * info.num_subcores
    rows_per_w = T // nw
    R = 16
    n_groups = rows_per_w // R

    mesh = plsc.VectorSubcoreMesh(core_axis_name="c", subcore_axis_name="s")

    @functools.partial(
        pl.kernel,
        mesh=mesh,
        out_type=jax.ShapeDtypeStruct((T, S, C_DIM), jnp.float32),
        scratch_types=[
            pltpu.VMEM((R, S), jnp.int32),
            pltpu.VMEM((R, S, C_DIM), jnp.float32),
            pltpu.SemaphoreType.DMA,
        ],
        compiler_params=pltpu.CompilerParams(use_tc_tiling_on_sc=False),
    )
    def k(tag_hbm, table_hbm, out_hbm, idx_v, rows_v, sem):
        wid = lax.axis_index("s") * info.num_cores + lax.axis_index("c")
        base = wid * rows_per_w

        def group(g, carry):
            r0 = base + g * R
            pltpu.sync_copy(tag_hbm.at[pl.ds(r0, R), :], idx_v)
            copies = [
                pltpu.async_copy(table_hbm.at[idx_v.at[j]], rows_v.at[j], sem)
                for j in range(R)
            ]
            for c in copies:
                c.wait()

            def srow(j, c2):
                def scol(s, c3):
                    rows_v[j, s, 0:16] = rows_v[j, s, 0:16] * SCALE
                    rows_v[j, s, 16:32] = rows_v[j, s, 16:32] * SCALE
                    return c3

                return lax.fori_loop(0, S, scol, c2, unroll=2)

            lax.fori_loop(0, R, srow, 0)
            pltpu.sync_copy(rows_v, out_hbm.at[pl.ds(r0, R)])
            return carry

        lax.fori_loop(0, n_groups, group, 0)

    return k(tag, table)


def kernel(tag, table):
    return _emb_lookup(tag.astype(jnp.int32), table)
